# baseline (speedup 1.0000x reference)
import functools

import jax
import jax.numpy as jnp
from jax import lax
from jax.experimental import pallas as pl
from jax.experimental.pallas import tpu as pltpu
from jax.experimental.pallas import tpu_sc as plsc

B, C, H, W = 4, 96, 384, 384
CP = 128
OUT_H, OUT_W = 384, 384
P = B * OUT_H * OUT_W
NC, NS, L = 2, 16, 16
NW = NC * NS
PPW = P // NW
CH = 96
NCHUNK = PPW // CH
NG = CH // L


def _interp_body(table_h, pg_h, out_h, *refs):
    pg = refs[0:2]
    IDX = (refs[2:6], refs[6:10])
    WTS = (refs[10:14], refs[14:18])
    ROWS = (refs[18:22], refs[22:26])
    ACC = refs[26:28]
    semg = refs[28:30]
    semc = refs[30:32]
    semo = refs[32:34]
    wid = lax.axis_index("s") * NC + lax.axis_index("c")
    rowb = lax.shift_right_logical(wid, 3) * (H * W)

    def coord_src(t):
        cid = wid * NCHUNK + t
        return pg_h.at[pl.ds(cid * (4 * CH), 4 * CH)]

    def fire(t, b):
        pltpu.make_async_copy(coord_src(t), pg[b], semc[b]).wait()
        ia_v, ib_v, ic_v, id_v = IDX[b]
        wa_v, wb_v, wc_v, wd_v = WTS[b]
        for v in range(NG):
            sl = pl.ds(v * L, L)
            x = pg[b][pl.ds(v * L, L)] + pg[b][pl.ds(2 * CH + v * L, L)]
            y = pg[b][pl.ds(CH + v * L, L)] + pg[b][pl.ds(3 * CH + v * L, L)]
            xi = (x + 1.0) * (W - 1.0) / 2.0
            yi = (y + 1.0) * (H - 1.0) / 2.0
            xt = xi.astype(jnp.int32)
            x0 = jnp.where(xt.astype(jnp.float32) > xi, xt - 1, xt)
            yt = yi.astype(jnp.int32)
            y0 = jnp.where(yt.astype(jnp.float32) > yi, yt - 1, yt)
            x0c = jnp.minimum(jnp.maximum(x0, 0), W - 2)
            x1c = jnp.minimum(jnp.maximum(x0 + 1, 0), W - 1)
            y0c = jnp.minimum(jnp.maximum(y0, 0), H - 2)
            y1c = jnp.minimum(jnp.maximum(y0 + 1, 0), H - 1)
            x0f = x0c.astype(jnp.float32)
            x1f = x1c.astype(jnp.float32)
            y0f = y0c.astype(jnp.float32)
            y1f = y1c.astype(jnp.float32)
            wa_v[sl] = (x1f - xi) * (y1f - yi)
            wb_v[sl] = (x1f - xi) * (yi - y0f)
            wc_v[sl] = (xi - x0f) * (y1f - yi)
            wd_v[sl] = (xi - x0f) * (yi - y0f)
            ia_v[sl] = rowb + y0c * W + x0c
            ib_v[sl] = rowb + y1c * W + x0c
            ic_v[sl] = rowb + y0c * W + x1c
            id_v[sl] = rowb + y1c * W + x1c
        for i in range(4):
            pltpu.async_copy(table_h.at[IDX[b][i]], ROWS[b][i], semg[b])

        @pl.when(t + 2 < NCHUNK)
        def _():
            pltpu.async_copy(coord_src(t + 2), pg[b], semc[b])

    def wait_gathers(b):
        for i in range(4):
            pltpu.make_async_copy(table_h.at[IDX[b][i]], ROWS[b][i],
                                  semg[b]).wait()

    def out_dst(t):
        return out_h.at[pl.ds((wid * PPW + t * CH) * C, CH * C)]

    def compute(t, b):
        wa_v, wb_v, wc_v, wd_v = WTS[b]
        ra_v, rb_v, rc_v, rd_v = ROWS[b]
        acc_v = ACC[b]

        @pl.when(t >= 2)
        def _():
            pltpu.make_async_copy(acc_v, out_dst(t - 2), semo[b]).wait()

        @plsc.parallel_loop(0, NG)
        def grp_body(v):
            gl = pl.ds(v * L, L)
            wa16 = wa_v[gl]
            wb16 = wb_v[gl]
            wc16 = wc_v[gl]
            wd16 = wd_v[gl]
            for k in range(L):
                p2 = v * L + k
                bwa = jnp.full((L,), wa16[k], jnp.float32)
                bwb = jnp.full((L,), wb16[k], jnp.float32)
                bwc = jnp.full((L,), wc16[k], jnp.float32)
                bwd = jnp.full((L,), wd16[k], jnp.float32)
                for g in range(C // L):
                    cs = pl.ds(g * L, L)
                    acc_v[pl.ds(p2 * C + g * L, L)] = (
                        bwa * ra_v[p2, cs] + bwb * rb_v[p2, cs]
                        + bwc * rc_v[p2, cs] + bwd * rd_v[p2, cs])

        pltpu.async_copy(acc_v, out_dst(t), semo[b])

    pltpu.async_copy(coord_src(0), pg[0], semc[0])
    pltpu.async_copy(coord_src(1), pg[1], semc[1])
    fire(0, 0)

    def pair_body(tp, _):
        t0 = 2 * tp
        fire(t0 + 1, 1)
        wait_gathers(0)
        compute(t0, 0)

        @pl.when(t0 + 2 < NCHUNK)
        def _():
            fire(t0 + 2, 0)

        wait_gathers(1)
        compute(t0 + 1, 1)
        return 0

    lax.fori_loop(0, NCHUNK // 2, pair_body, 0)
    pltpu.make_async_copy(ACC[0], out_dst(NCHUNK - 2), semo[0]).wait()
    pltpu.make_async_copy(ACC[1], out_dst(NCHUNK - 1), semo[1]).wait()


@jax.jit
def _sc_interp(table, pgrid):
    mesh = plsc.VectorSubcoreMesh(core_axis_name="c", subcore_axis_name="s")
    vf = lambda n, d: [pltpu.VMEM((n,), d)] * 4
    f = functools.partial(
        pl.kernel,
        out_type=jax.ShapeDtypeStruct((P * C,), jnp.float32),
        mesh=mesh,
        scratch_types=(
            [pltpu.VMEM((4 * CH,), jnp.float32)] * 2
            + vf(CH, jnp.int32) + vf(CH, jnp.int32)
            + vf(CH, jnp.float32) + vf(CH, jnp.float32)
            + [pltpu.VMEM((CH, CP), jnp.float32)] * 8
            + [pltpu.VMEM((CH * C,), jnp.float32)] * 2
            + [pltpu.SemaphoreType.DMA] * 6
        ),
    )(_interp_body)
    return f(table, pgrid)


def kernel(U, theta, out_size):
    dep = jnp.asarray((out_size[0] - OUT_H) + (out_size[1] - OUT_W), jnp.float32)
    table = jnp.pad(
        jnp.transpose(U, (0, 2, 3, 1)).reshape(B * H * W, C),
        ((0, 0), (0, CP - C)))
    tx = theta[:, 0, :].astype(jnp.float32).reshape(-1)
    ty = theta[:, 1, :].astype(jnp.float32).reshape(-1)
    lin_x = jnp.linspace(-1.0, 1.0, OUT_W, dtype=jnp.float32)
    lin_y = jnp.linspace(-1.0, 1.0, OUT_H, dtype=jnp.float32)
    gx = jnp.tile(lin_x, B * OUT_H) + dep
    gy = jnp.tile(jnp.repeat(lin_y, OUT_W), B) + dep
    pgrid = jnp.stack([tx, ty, gx, gy], 0).reshape(4, P // CH, CH)
    pgrid = jnp.transpose(pgrid, (1, 0, 2)).reshape(-1)
    out_flat = _sc_interp(table, pgrid)
    return jnp.transpose(out_flat.reshape(B, OUT_H, OUT_W, C), (0, 3, 1, 2))


# speedup vs baseline: 1.0020x; 1.0020x over previous
import functools

import jax
import jax.numpy as jnp
from jax import lax
from jax.experimental import pallas as pl
from jax.experimental.pallas import tpu as pltpu
from jax.experimental.pallas import tpu_sc as plsc

B, C, H, W = 4, 96, 384, 384
CP = 128
OUT_H, OUT_W = 384, 384
P = B * OUT_H * OUT_W
NC, NS, L = 2, 16, 16
NW = NC * NS
PPW = P // NW
CH = 96
NCHUNK = PPW // CH
NG = CH // L
PROBE_TRIVIAL_BLEND = False


def _interp_body(table_h, pg_h, out_h, *refs):
    pg = refs[0:2]
    IDX = (refs[2:6], refs[6:10])
    WTS = (refs[10:14], refs[14:18])
    ROWS = (refs[18:22], refs[22:26])
    ACC = refs[26:28]
    semg = refs[28:30]
    semc = refs[30:32]
    semo = refs[32:34]
    wid = lax.axis_index("s") * NC + lax.axis_index("c")
    rowb = lax.shift_right_logical(wid, 3) * (H * W)

    def coord_src(t):
        cid = wid * NCHUNK + t
        return pg_h.at[pl.ds(cid * (4 * CH), 4 * CH)]

    def fire(t, b):
        pltpu.make_async_copy(coord_src(t), pg[b], semc[b]).wait()
        ia_v, ib_v, ic_v, id_v = IDX[b]
        wa_v, wb_v, wc_v, wd_v = WTS[b]
        for v in range(NG):
            sl = pl.ds(v * L, L)
            x = pg[b][pl.ds(v * L, L)] + pg[b][pl.ds(2 * CH + v * L, L)]
            y = pg[b][pl.ds(CH + v * L, L)] + pg[b][pl.ds(3 * CH + v * L, L)]
            xi = (x + 1.0) * (W - 1.0) / 2.0
            yi = (y + 1.0) * (H - 1.0) / 2.0
            xt = xi.astype(jnp.int32)
            x0 = jnp.where(xt.astype(jnp.float32) > xi, xt - 1, xt)
            yt = yi.astype(jnp.int32)
            y0 = jnp.where(yt.astype(jnp.float32) > yi, yt - 1, yt)
            x0c = jnp.minimum(jnp.maximum(x0, 0), W - 2)
            x1c = jnp.minimum(jnp.maximum(x0 + 1, 0), W - 1)
            y0c = jnp.minimum(jnp.maximum(y0, 0), H - 2)
            y1c = jnp.minimum(jnp.maximum(y0 + 1, 0), H - 1)
            x0f = x0c.astype(jnp.float32)
            x1f = x1c.astype(jnp.float32)
            y0f = y0c.astype(jnp.float32)
            y1f = y1c.astype(jnp.float32)
            wa_v[sl] = (x1f - xi) * (y1f - yi)
            wb_v[sl] = (x1f - xi) * (yi - y0f)
            wc_v[sl] = (xi - x0f) * (y1f - yi)
            wd_v[sl] = (xi - x0f) * (yi - y0f)
            ia_v[sl] = rowb + y0c * W + x0c
            ib_v[sl] = rowb + y1c * W + x0c
            ic_v[sl] = rowb + y0c * W + x1c
            id_v[sl] = rowb + y1c * W + x1c
        for i in range(4):
            pltpu.async_copy(table_h.at[IDX[b][i]], ROWS[b][i], semg[b])

        @pl.when(t + 2 < NCHUNK)
        def _():
            pltpu.async_copy(coord_src(t + 2), pg[b], semc[b])

    def wait_gathers(b):
        for i in range(4):
            pltpu.make_async_copy(table_h.at[IDX[b][i]], ROWS[b][i],
                                  semg[b]).wait()

    def out_dst(t):
        return out_h.at[pl.ds((wid * PPW + t * CH) * C, CH * C)]

    def compute(t, b):
        wa_v, wb_v, wc_v, wd_v = WTS[b]
        ra_v, rb_v, rc_v, rd_v = ROWS[b]
        acc_v = ACC[b]

        @pl.when(t >= 2)
        def _():
            pltpu.make_async_copy(acc_v, out_dst(t - 2), semo[b]).wait()

        @plsc.parallel_loop(0, NG)
        def grp_body(v):
            gl = pl.ds(v * L, L)
            wa16 = wa_v[gl]
            wb16 = wb_v[gl]
            wc16 = wc_v[gl]
            wd16 = wd_v[gl]
            for k in range(L):
                p2 = v * L + k
                if PROBE_TRIVIAL_BLEND:
                    for g in range(C // L):
                        cs = pl.ds(g * L, L)
                        acc_v[pl.ds(p2 * C + g * L, L)] = ra_v[p2, cs]
                else:
                    bwa = jnp.full((L,), wa16[k], jnp.float32)
                    bwb = jnp.full((L,), wb16[k], jnp.float32)
                    bwc = jnp.full((L,), wc16[k], jnp.float32)
                    bwd = jnp.full((L,), wd16[k], jnp.float32)
                    for g in range(C // L):
                        cs = pl.ds(g * L, L)
                        acc_v[pl.ds(p2 * C + g * L, L)] = (
                            bwa * ra_v[p2, cs] + bwb * rb_v[p2, cs]
                            + bwc * rc_v[p2, cs] + bwd * rd_v[p2, cs])

        pltpu.async_copy(acc_v, out_dst(t), semo[b])

    pltpu.async_copy(coord_src(0), pg[0], semc[0])
    pltpu.async_copy(coord_src(1), pg[1], semc[1])
    fire(0, 0)

    def pair_body(tp, _):
        t0 = 2 * tp
        fire(t0 + 1, 1)
        wait_gathers(0)
        compute(t0, 0)

        @pl.when(t0 + 2 < NCHUNK)
        def _():
            fire(t0 + 2, 0)

        wait_gathers(1)
        compute(t0 + 1, 1)
        return 0

    lax.fori_loop(0, NCHUNK // 2, pair_body, 0)
    pltpu.make_async_copy(ACC[0], out_dst(NCHUNK - 2), semo[0]).wait()
    pltpu.make_async_copy(ACC[1], out_dst(NCHUNK - 1), semo[1]).wait()


@jax.jit
def _sc_interp(table, pgrid):
    mesh = plsc.VectorSubcoreMesh(core_axis_name="c", subcore_axis_name="s")
    vf = lambda n, d: [pltpu.VMEM((n,), d)] * 4
    f = functools.partial(
        pl.kernel,
        out_type=jax.ShapeDtypeStruct((P * C,), jnp.float32),
        mesh=mesh,
        scratch_types=(
            [pltpu.VMEM((4 * CH,), jnp.float32)] * 2
            + vf(CH, jnp.int32) + vf(CH, jnp.int32)
            + vf(CH, jnp.float32) + vf(CH, jnp.float32)
            + [pltpu.VMEM((CH, CP), jnp.float32)] * 8
            + [pltpu.VMEM((CH * C,), jnp.float32)] * 2
            + [pltpu.SemaphoreType.DMA] * 6
        ),
    )(_interp_body)
    return f(table, pgrid)


def kernel(U, theta, out_size):
    dep = jnp.asarray((out_size[0] - OUT_H) + (out_size[1] - OUT_W), jnp.float32)
    table = jnp.pad(
        jnp.transpose(U, (0, 2, 3, 1)).reshape(B * H * W, C),
        ((0, 0), (0, CP - C)))
    tx = theta[:, 0, :].astype(jnp.float32).reshape(-1)
    ty = theta[:, 1, :].astype(jnp.float32).reshape(-1)
    lin_x = jnp.linspace(-1.0, 1.0, OUT_W, dtype=jnp.float32)
    lin_y = jnp.linspace(-1.0, 1.0, OUT_H, dtype=jnp.float32)
    gx = jnp.tile(lin_x, B * OUT_H) + dep
    gy = jnp.tile(jnp.repeat(lin_y, OUT_W), B) + dep
    pgrid = jnp.stack([tx, ty, gx, gy], 0).reshape(4, P // CH, CH)
    pgrid = jnp.transpose(pgrid, (1, 0, 2)).reshape(-1)
    out_flat = _sc_interp(table, pgrid)
    return jnp.transpose(out_flat.reshape(B, OUT_H, OUT_W, C), (0, 3, 1, 2))
